# baseline (speedup 1.0000x reference)
import jax
import jax.numpy as jnp
from jax.experimental import pallas as pl

N = 10000
E = 320000
H = 128
OUT = 64
EPS = 1e-5


def _gcn(x, src, dst, W, b):
    n = x.shape[0]
    loop = jnp.arange(n, dtype=src.dtype)
    s = jnp.concatenate([src, loop])
    d = jnp.concatenate([dst, loop])
    deg = jnp.zeros((n,), dtype=x.dtype).at[d].add(1.0)
    dinv = jax.lax.rsqrt(jnp.maximum(deg, 1.0))
    norm = dinv[s] * dinv[d]
    h = x @ W
    msg = h[s] * norm[:, None]
    out = jax.ops.segment_sum(msg, d, num_segments=n)
    return out + b


def _final_body(h_ref, wf_ref, bf_ref, o_ref):
    o = h_ref[...] @ wf_ref[...] + bf_ref[...]
    m = jnp.max(o, axis=1, keepdims=True)
    lse = jnp.log(jnp.sum(jnp.exp(o - m), axis=1, keepdims=True)) + m
    o_ref[...] = o - lse


def kernel(x, edge_index, W_proj0, b_proj0, W_proj1, b_proj1, alpha, W1, b1, g1, be1, W2, b2, g2, be2, Wf, bf):
    src, dst = edge_index[0], edge_index[1]
    aw = jax.nn.softmax(alpha)
    x0 = x[:, 0:64] @ W_proj0 + b_proj0
    x1 = x[:, 64:128] @ W_proj1 + b_proj1
    xf = aw[0] * x0 + aw[1] * x1
    gs1 = g1 / jnp.sqrt(1.0 + EPS)
    gs2 = g2 / jnp.sqrt(1.0 + EPS)
    h = jax.nn.relu(_gcn(xf, src, dst, W1, b1) * gs1 + be1)
    h = jax.nn.relu(_gcn(h, src, dst, W2, b2) * gs2 + be2)
    out = pl.pallas_call(
        _final_body,
        out_shape=jax.ShapeDtypeStruct((N, OUT), jnp.float32),
        grid=(10,),
        in_specs=[
            pl.BlockSpec((N // 10, H), lambda i: (i, 0)),
            pl.BlockSpec((H, OUT), lambda i: (0, 0)),
            pl.BlockSpec((OUT,), lambda i: (0,)),
        ],
        out_specs=pl.BlockSpec((N // 10, OUT), lambda i: (i, 0)),
    )(h, Wf, bf)
    return out


# speedup vs baseline: 27.9341x; 27.9341x over previous
import functools

import jax
import jax.numpy as jnp
from jax import lax
from jax.experimental import pallas as pl
from jax.experimental.pallas import tpu as pltpu
from jax.experimental.pallas import tpu_sc as plsc

N = 10000
E = 320000
H = 128
OUT = 64
EPS = 1e-5

NSC = 2
NTEC = 16
NW = NSC * NTEC
K = 125
NCH = E // (NW * K)
CPG = 16
NG = NCH // CPG
NP = 10240
ZR = NP // NTEC
ZB = 80
EPT = E // NW

_mesh = plsc.VectorSubcoreMesh(core_axis_name="c", subcore_axis_name="s")


@functools.partial(
    pl.kernel,
    mesh=_mesh,
    out_type=jax.ShapeDtypeStruct((NW, N), jnp.float32),
    scratch_types=[
        pltpu.VMEM((EPT,), jnp.int32),
        pltpu.VMEM((N,), jnp.float32),
    ],
    compiler_params=pltpu.CompilerParams(needs_layout_passes=False),
)
def _deg_kernel(dst_hbm, out_hbm, dv, hist):
    c = lax.axis_index("c")
    s = lax.axis_index("s")
    w = c * NTEC + s
    pltpu.sync_copy(dst_hbm.at[pl.ds(w * EPT, EPT)], dv)
    zeros = jnp.zeros((16,), jnp.float32)

    def zbody(i, _):
        hist[pl.ds(i * 16, 16)] = zeros
        return 0

    lax.fori_loop(0, N // 16, zbody, 0, unroll=8)
    ones = jnp.ones((16,), jnp.float32)

    def body(i, _):
        idx = dv[pl.ds(i * 16, 16)]
        plsc.addupdate_scatter(hist, [idx], ones)
        return 0

    lax.fori_loop(0, EPT // 16, body, 0, unroll=8)
    pltpu.sync_copy(hist, out_hbm.at[w])


@functools.partial(
    pl.kernel,
    mesh=_mesh,
    out_type=jax.ShapeDtypeStruct((NSC, NP, H), jnp.float32),
    scratch_types=[
        pltpu.VMEM((CPG, K), jnp.int32),
        pltpu.VMEM((CPG, K), jnp.int32),
        pltpu.VMEM((2, K, H), jnp.float32),
        pltpu.VMEM((ZB, H), jnp.float32),
        pltpu.VMEM_SHARED((NP, H), jnp.float32),
        pltpu.SemaphoreType.DMA,
        pltpu.SemaphoreType.DMA,
    ],
    compiler_params=pltpu.CompilerParams(needs_layout_passes=False),
)
def _scatter_kernel(table_hbm, src_hbm, dst_hbm, z_hbm, out_hbm, src_v, dst_v, rows, zb, acc, g0, g1):
    c = lax.axis_index("c")
    s = lax.axis_index("s")
    w = c * NTEC + s

    pltpu.sync_copy(z_hbm, zb)
    for k in range(ZR // ZB):
        pltpu.sync_copy(zb, acc.at[pl.ds(s * ZR + k * ZB, ZB)])
    plsc.subcore_barrier()

    def _gather(j, buf, sem):
        pltpu.async_copy(table_hbm.at[src_v.at[j]], rows.at[buf], sem)

    def _gwait(buf, sem):
        pltpu.make_async_copy(table_hbm.at[src_v.at[0]], rows.at[buf], sem).wait()

    def gbody(g, _):
        pltpu.sync_copy(src_hbm.at[w, pl.ds(g * CPG, CPG)], src_v)
        pltpu.sync_copy(dst_hbm.at[w, pl.ds(g * CPG, CPG)], dst_v)
        _gather(0, 0, g0)

        def mbody(i2, _):
            j0 = 2 * i2
            j1 = j0 + 1
            _gather(j1, 1, g1)
            _gwait(0, g0)
            pltpu.sync_copy(rows.at[0], acc.at[dst_v.at[j0]], add=True)

            @pl.when(j0 + 2 < CPG)
            def _():
                _gather(j0 + 2, 0, g0)

            _gwait(1, g1)
            pltpu.sync_copy(rows.at[1], acc.at[dst_v.at[j1]], add=True)
            return 0

        lax.fori_loop(0, CPG // 2, mbody, 0)
        return 0

    lax.fori_loop(0, NG, gbody, 0)
    plsc.subcore_barrier()
    for k in range(ZR // ZB):
        pltpu.sync_copy(acc.at[pl.ds(s * ZR + k * ZB, ZB)], zb)
        pltpu.sync_copy(zb, out_hbm.at[c, pl.ds(s * ZR + k * ZB, ZB)])


def _dinv_body(dp_ref, o_ref):
    deg = 1.0 + jnp.sum(dp_ref[...], axis=0)
    o_ref[...] = lax.rsqrt(jnp.maximum(deg, 1.0))[:, None]


def _proj_body(x_ref, wc_ref, bc_ref, w1_ref, dv_ref, o_ref):
    xf = x_ref[...] @ wc_ref[...] + bc_ref[...]
    h1 = xf @ w1_ref[...]
    o_ref[...] = h1 * dv_ref[...]


def _mid_body(yp_ref, ht_ref, dv_ref, s_ref, t_ref, w2_ref, o_ref):
    dinv = dv_ref[...]
    agg = yp_ref[0] + yp_ref[1] + ht_ref[...]
    z = jnp.maximum(agg * dinv * s_ref[...] + t_ref[...], 0.0)
    o_ref[...] = (z @ w2_ref[...]) * dinv


def _final_body(yp_ref, ht_ref, dv_ref, s_ref, t_ref, wf_ref, bf_ref, o_ref):
    dinv = dv_ref[...]
    agg = yp_ref[0] + yp_ref[1] + ht_ref[...]
    z = jnp.maximum(agg * dinv * s_ref[...] + t_ref[...], 0.0)
    o = z @ wf_ref[...] + bf_ref[...]
    m = jnp.max(o, axis=1, keepdims=True)
    lse = jnp.log(jnp.sum(jnp.exp(o - m), axis=1, keepdims=True)) + m
    o_ref[...] = o - lse


_GRID = 10
_BLK = N // _GRID


def _row_spec(width):
    return pl.BlockSpec((_BLK, width), lambda i: (i, 0))


def _full_spec(r, cdim):
    return pl.BlockSpec((r, cdim), lambda i: (0, 0))


def kernel(x, edge_index, W_proj0, b_proj0, W_proj1, b_proj1, alpha, W1, b1, g1, be1, W2, b2, g2, be2, Wf, bf):
    aw = jax.nn.softmax(alpha)
    Wc = jnp.concatenate([aw[0] * W_proj0, aw[1] * W_proj1], axis=0)
    bc = (aw[0] * b_proj0 + aw[1] * b_proj1).reshape(1, H)
    gs1 = (g1 / jnp.sqrt(1.0 + EPS)).reshape(1, H)
    t1 = (b1 * gs1[0] + be1).reshape(1, H)
    gs2 = (g2 / jnp.sqrt(1.0 + EPS)).reshape(1, H)
    t2 = (b2 * gs2[0] + be2).reshape(1, H)
    bf2 = bf.reshape(1, OUT)
    src_r = edge_index[0].reshape(NW, NCH, K)
    dst_r = edge_index[1].reshape(NW, NCH, K)
    dst1d = edge_index[1]
    zbuf = jnp.zeros((ZB, H), jnp.float32)

    dp = _deg_kernel(dst1d)

    dv = pl.pallas_call(
        _dinv_body,
        out_shape=jax.ShapeDtypeStruct((N, 1), jnp.float32),
        grid=(1,),
        in_specs=[pl.BlockSpec((NW, N), lambda i: (0, 0))],
        out_specs=pl.BlockSpec((N, 1), lambda i: (0, 0)),
    )(dp)

    ht1 = pl.pallas_call(
        _proj_body,
        out_shape=jax.ShapeDtypeStruct((N, H), jnp.float32),
        grid=(_GRID,),
        in_specs=[
            _row_spec(H),
            _full_spec(H, H),
            _full_spec(1, H),
            _full_spec(H, H),
            _row_spec(1),
        ],
        out_specs=_row_spec(H),
    )(x, Wc, bc, W1, dv)

    yp1 = _scatter_kernel(ht1, src_r, dst_r, zbuf)

    ht2 = pl.pallas_call(
        _mid_body,
        out_shape=jax.ShapeDtypeStruct((N, H), jnp.float32),
        grid=(_GRID,),
        in_specs=[
            pl.BlockSpec((NSC, _BLK, H), lambda i: (0, i, 0)),
            _row_spec(H),
            _row_spec(1),
            _full_spec(1, H),
            _full_spec(1, H),
            _full_spec(H, H),
        ],
        out_specs=_row_spec(H),
    )(yp1, ht1, dv, gs1, t1, W2)

    yp2 = _scatter_kernel(ht2, src_r, dst_r, zbuf)

    out = pl.pallas_call(
        _final_body,
        out_shape=jax.ShapeDtypeStruct((N, OUT), jnp.float32),
        grid=(_GRID,),
        in_specs=[
            pl.BlockSpec((NSC, _BLK, H), lambda i: (0, i, 0)),
            _row_spec(H),
            _row_spec(1),
            _full_spec(1, H),
            _full_spec(1, H),
            _full_spec(H, OUT),
            _full_spec(1, OUT),
        ],
        out_specs=pl.BlockSpec((_BLK, OUT), lambda i: (i, 0)),
    )(yp2, ht2, dv, gs2, t2, Wf, bf2)
    return out


# speedup vs baseline: 27.9681x; 1.0012x over previous
import functools

import jax
import jax.numpy as jnp
from jax import lax
from jax.experimental import pallas as pl
from jax.experimental.pallas import tpu as pltpu
from jax.experimental.pallas import tpu_sc as plsc

N = 10000
E = 320000
H = 128
OUT = 64
EPS = 1e-5

NSC = 2
NTEC = 16
NW = NSC * NTEC
K = 125
NCH = E // (NW * K)
CPG = 16
NG = NCH // CPG
NP = 10240
ZR = NP // NTEC
ZB = 80
EPT = E // NW

_mesh = plsc.VectorSubcoreMesh(core_axis_name="c", subcore_axis_name="s")


@functools.partial(
    pl.kernel,
    mesh=_mesh,
    out_type=jax.ShapeDtypeStruct((NW, N), jnp.float32),
    scratch_types=[
        pltpu.VMEM((EPT,), jnp.int32),
        pltpu.VMEM((N,), jnp.float32),
    ],
    compiler_params=pltpu.CompilerParams(needs_layout_passes=False),
)
def _deg_kernel(dst_hbm, out_hbm, dv, hist):
    c = lax.axis_index("c")
    s = lax.axis_index("s")
    w = c * NTEC + s
    pltpu.sync_copy(dst_hbm.at[pl.ds(w * EPT, EPT)], dv)
    zeros = jnp.zeros((16,), jnp.float32)

    def zbody(i, _):
        hist[pl.ds(i * 16, 16)] = zeros
        return 0

    lax.fori_loop(0, N // 16, zbody, 0, unroll=8)
    ones = jnp.ones((16,), jnp.float32)

    def body(i, _):
        idx = dv[pl.ds(i * 16, 16)]
        plsc.addupdate_scatter(hist, [idx], ones)
        return 0

    lax.fori_loop(0, EPT // 16, body, 0, unroll=8)
    pltpu.sync_copy(hist, out_hbm.at[w])


@functools.partial(
    pl.kernel,
    mesh=_mesh,
    out_type=jax.ShapeDtypeStruct((NSC, NP, H), jnp.float32),
    scratch_types=[
        pltpu.VMEM((CPG, K), jnp.int32),
        pltpu.VMEM((CPG, K), jnp.int32),
        pltpu.VMEM((2, K, H), jnp.float32),
        pltpu.VMEM((ZB, H), jnp.float32),
        pltpu.VMEM_SHARED((NP, H), jnp.float32),
        pltpu.SemaphoreType.DMA,
        pltpu.SemaphoreType.DMA,
        pltpu.SemaphoreType.DMA,
        pltpu.SemaphoreType.DMA,
    ],
    compiler_params=pltpu.CompilerParams(needs_layout_passes=False),
)
def _scatter_kernel(table_hbm, src_hbm, dst_hbm, z_hbm, out_hbm, src_v, dst_v, rows, zb, acc, g0, g1, s0, s1):
    c = lax.axis_index("c")
    s = lax.axis_index("s")
    w = c * NTEC + s

    pltpu.sync_copy(z_hbm, zb)
    for k in range(ZR // ZB):
        pltpu.sync_copy(zb, acc.at[pl.ds(s * ZR + k * ZB, ZB)])
    plsc.subcore_barrier()

    def _gather(j, buf, sem):
        pltpu.async_copy(table_hbm.at[src_v.at[j]], rows.at[buf], sem)

    def _gwait(buf, sem):
        pltpu.make_async_copy(table_hbm.at[src_v.at[0]], rows.at[buf], sem).wait()

    def _scat(j, buf, sem):
        pltpu.async_copy(rows.at[buf], acc.at[dst_v.at[j]], sem, add=True)

    def _swait(buf, sem):
        pltpu.make_async_copy(rows.at[buf], acc.at[dst_v.at[0]], sem).wait()

    def gbody(g, _):
        pltpu.sync_copy(src_hbm.at[w, pl.ds(g * CPG, CPG)], src_v)
        pltpu.sync_copy(dst_hbm.at[w, pl.ds(g * CPG, CPG)], dst_v)
        _gather(0, 0, g0)

        def mbody(i2, _):
            j0 = 2 * i2
            j1 = j0 + 1

            @pl.when(i2 > 0)
            def _():
                _swait(1, s1)

            _gather(j1, 1, g1)
            _gwait(0, g0)
            _scat(j0, 0, s0)

            @pl.when(j0 + 2 < CPG)
            def _():
                _swait(0, s0)
                _gather(j0 + 2, 0, g0)

            _gwait(1, g1)
            _scat(j1, 1, s1)
            return 0

        lax.fori_loop(0, CPG // 2, mbody, 0)
        _swait(0, s0)
        _swait(1, s1)
        return 0

    lax.fori_loop(0, NG, gbody, 0)
    plsc.subcore_barrier()
    for k in range(ZR // ZB):
        pltpu.sync_copy(acc.at[pl.ds(s * ZR + k * ZB, ZB)], zb)
        pltpu.sync_copy(zb, out_hbm.at[c, pl.ds(s * ZR + k * ZB, ZB)])


def _dinv_body(dp_ref, o_ref):
    deg = 1.0 + jnp.sum(dp_ref[...], axis=0)
    o_ref[...] = lax.rsqrt(jnp.maximum(deg, 1.0))[:, None]


def _proj_body(x_ref, wc_ref, bc_ref, w1_ref, dv_ref, o_ref):
    xf = x_ref[...] @ wc_ref[...] + bc_ref[...]
    h1 = xf @ w1_ref[...]
    o_ref[...] = h1 * dv_ref[...]


def _mid_body(yp_ref, ht_ref, dv_ref, s_ref, t_ref, w2_ref, o_ref):
    dinv = dv_ref[...]
    agg = yp_ref[0] + yp_ref[1] + ht_ref[...]
    z = jnp.maximum(agg * dinv * s_ref[...] + t_ref[...], 0.0)
    o_ref[...] = (z @ w2_ref[...]) * dinv


def _final_body(yp_ref, ht_ref, dv_ref, s_ref, t_ref, wf_ref, bf_ref, o_ref):
    dinv = dv_ref[...]
    agg = yp_ref[0] + yp_ref[1] + ht_ref[...]
    z = jnp.maximum(agg * dinv * s_ref[...] + t_ref[...], 0.0)
    o = z @ wf_ref[...] + bf_ref[...]
    m = jnp.max(o, axis=1, keepdims=True)
    lse = jnp.log(jnp.sum(jnp.exp(o - m), axis=1, keepdims=True)) + m
    o_ref[...] = o - lse


_GRID = 10
_BLK = N // _GRID


def _row_spec(width):
    return pl.BlockSpec((_BLK, width), lambda i: (i, 0))


def _full_spec(r, cdim):
    return pl.BlockSpec((r, cdim), lambda i: (0, 0))


def kernel(x, edge_index, W_proj0, b_proj0, W_proj1, b_proj1, alpha, W1, b1, g1, be1, W2, b2, g2, be2, Wf, bf):
    aw = jax.nn.softmax(alpha)
    Wc = jnp.concatenate([aw[0] * W_proj0, aw[1] * W_proj1], axis=0)
    bc = (aw[0] * b_proj0 + aw[1] * b_proj1).reshape(1, H)
    gs1 = (g1 / jnp.sqrt(1.0 + EPS)).reshape(1, H)
    t1 = (b1 * gs1[0] + be1).reshape(1, H)
    gs2 = (g2 / jnp.sqrt(1.0 + EPS)).reshape(1, H)
    t2 = (b2 * gs2[0] + be2).reshape(1, H)
    bf2 = bf.reshape(1, OUT)
    src_r = edge_index[0].reshape(NW, NCH, K)
    dst_r = edge_index[1].reshape(NW, NCH, K)
    dst1d = edge_index[1]
    zbuf = jnp.zeros((ZB, H), jnp.float32)

    dp = _deg_kernel(dst1d)

    dv = pl.pallas_call(
        _dinv_body,
        out_shape=jax.ShapeDtypeStruct((N, 1), jnp.float32),
        grid=(1,),
        in_specs=[pl.BlockSpec((NW, N), lambda i: (0, 0))],
        out_specs=pl.BlockSpec((N, 1), lambda i: (0, 0)),
    )(dp)

    ht1 = pl.pallas_call(
        _proj_body,
        out_shape=jax.ShapeDtypeStruct((N, H), jnp.float32),
        grid=(_GRID,),
        in_specs=[
            _row_spec(H),
            _full_spec(H, H),
            _full_spec(1, H),
            _full_spec(H, H),
            _row_spec(1),
        ],
        out_specs=_row_spec(H),
    )(x, Wc, bc, W1, dv)

    yp1 = _scatter_kernel(ht1, src_r, dst_r, zbuf)

    ht2 = pl.pallas_call(
        _mid_body,
        out_shape=jax.ShapeDtypeStruct((N, H), jnp.float32),
        grid=(_GRID,),
        in_specs=[
            pl.BlockSpec((NSC, _BLK, H), lambda i: (0, i, 0)),
            _row_spec(H),
            _row_spec(1),
            _full_spec(1, H),
            _full_spec(1, H),
            _full_spec(H, H),
        ],
        out_specs=_row_spec(H),
    )(yp1, ht1, dv, gs1, t1, W2)

    yp2 = _scatter_kernel(ht2, src_r, dst_r, zbuf)

    out = pl.pallas_call(
        _final_body,
        out_shape=jax.ShapeDtypeStruct((N, OUT), jnp.float32),
        grid=(_GRID,),
        in_specs=[
            pl.BlockSpec((NSC, _BLK, H), lambda i: (0, i, 0)),
            _row_spec(H),
            _row_spec(1),
            _full_spec(1, H),
            _full_spec(1, H),
            _full_spec(H, OUT),
            _full_spec(1, OUT),
        ],
        out_specs=pl.BlockSpec((_BLK, OUT), lambda i: (i, 0)),
    )(yp2, ht2, dv, gs2, t2, Wf, bf2)
    return out


# speedup vs baseline: 29.9217x; 1.0699x over previous
import functools

import jax
import jax.numpy as jnp
from jax import lax
from jax.experimental import pallas as pl
from jax.experimental.pallas import tpu as pltpu
from jax.experimental.pallas import tpu_sc as plsc

N = 10000
E = 320000
H = 128
OUT = 64
EPS = 1e-5

NSC = 2
NTEC = 16
NW = NSC * NTEC
K = 100
NCH = E // (NW * K)
CPG = 20
NG = NCH // CPG
RING = 3
NP = 10240
ZR = NP // NTEC
ZB = 80
EPT = E // NW

_mesh = plsc.VectorSubcoreMesh(core_axis_name="c", subcore_axis_name="s")


@functools.partial(
    pl.kernel,
    mesh=_mesh,
    out_type=jax.ShapeDtypeStruct((NW, N), jnp.float32),
    scratch_types=[
        pltpu.VMEM((EPT,), jnp.int32),
        pltpu.VMEM((N,), jnp.float32),
    ],
    compiler_params=pltpu.CompilerParams(needs_layout_passes=False),
)
def _deg_kernel(dst_hbm, out_hbm, dv, hist):
    c = lax.axis_index("c")
    s = lax.axis_index("s")
    w = c * NTEC + s
    pltpu.sync_copy(dst_hbm.at[pl.ds(w * EPT, EPT)], dv)
    zeros = jnp.zeros((16,), jnp.float32)

    def zbody(i, _):
        hist[pl.ds(i * 16, 16)] = zeros
        return 0

    lax.fori_loop(0, N // 16, zbody, 0, unroll=8)
    ones = jnp.ones((16,), jnp.float32)

    def body(i, _):
        idx = dv[pl.ds(i * 16, 16)]
        plsc.addupdate_scatter(hist, [idx], ones)
        return 0

    lax.fori_loop(0, EPT // 16, body, 0, unroll=8)
    pltpu.sync_copy(hist, out_hbm.at[w])


@functools.partial(
    pl.kernel,
    mesh=_mesh,
    out_type=jax.ShapeDtypeStruct((NSC, NP, H), jnp.float32),
    scratch_types=[
        pltpu.VMEM((CPG, K), jnp.int32),
        pltpu.VMEM((CPG, K), jnp.int32),
        pltpu.VMEM((RING, K, H), jnp.float32),
        pltpu.VMEM_SHARED((NP, H), jnp.float32),
        pltpu.SemaphoreType.DMA((RING,)),
        pltpu.SemaphoreType.DMA((RING,)),
    ],
    compiler_params=pltpu.CompilerParams(needs_layout_passes=False),
)
def _scatter_kernel(table_hbm, src_hbm, dst_hbm, z_hbm, out_hbm, src_v, dst_v, rows, acc, gsem, ssem):
    c = lax.axis_index("c")
    s = lax.axis_index("s")
    w = c * NTEC + s

    pltpu.sync_copy(z_hbm, rows.at[0, pl.ds(0, ZB)])
    for k in range(ZR // ZB):
        pltpu.sync_copy(rows.at[0, pl.ds(0, ZB)], acc.at[pl.ds(s * ZR + k * ZB, ZB)])
    plsc.subcore_barrier()

    def _gather(j, buf):
        pltpu.async_copy(table_hbm.at[src_v.at[j]], rows.at[buf], gsem.at[buf])

    def _gwait(buf):
        pltpu.make_async_copy(table_hbm.at[src_v.at[0]], rows.at[buf], gsem.at[buf]).wait()

    def _scat(j, buf):
        pltpu.async_copy(rows.at[buf], acc.at[dst_v.at[j]], ssem.at[buf], add=True)

    def _swait(buf):
        pltpu.make_async_copy(rows.at[buf], acc.at[dst_v.at[0]], ssem.at[buf]).wait()

    def gbody(g, _):
        pltpu.sync_copy(src_hbm.at[w, g], src_v)
        pltpu.sync_copy(dst_hbm.at[w, g], dst_v)
        for b in range(RING):
            _gather(b, b)

        def jbody(j, _):
            b = lax.rem(j, RING)
            _gwait(b)
            _scat(j, b)

            @pl.when(j + RING < CPG)
            def _():
                _swait(b)
                _gather(j + RING, b)

            return 0

        lax.fori_loop(0, CPG, jbody, 0)
        for o in range(RING):
            _swait((CPG - RING + o) % RING)
        return 0

    lax.fori_loop(0, NG, gbody, 0)
    plsc.subcore_barrier()
    for k in range(ZR // ZB):
        pltpu.sync_copy(acc.at[pl.ds(s * ZR + k * ZB, ZB)], rows.at[0, pl.ds(0, ZB)])
        pltpu.sync_copy(rows.at[0, pl.ds(0, ZB)], out_hbm.at[c, pl.ds(s * ZR + k * ZB, ZB)])


def _dinv_body(dp_ref, o_ref):
    deg = 1.0 + jnp.sum(dp_ref[...], axis=0)
    o_ref[...] = lax.rsqrt(jnp.maximum(deg, 1.0))[:, None]


def _proj_body(x_ref, wc_ref, bc_ref, w1_ref, dv_ref, o_ref):
    xf = x_ref[...] @ wc_ref[...] + bc_ref[...]
    h1 = xf @ w1_ref[...]
    o_ref[...] = h1 * dv_ref[...]


def _mid_body(yp_ref, ht_ref, dv_ref, s_ref, t_ref, w2_ref, o_ref):
    dinv = dv_ref[...]
    agg = yp_ref[0] + yp_ref[1] + ht_ref[...]
    z = jnp.maximum(agg * dinv * s_ref[...] + t_ref[...], 0.0)
    o_ref[...] = (z @ w2_ref[...]) * dinv


def _final_body(yp_ref, ht_ref, dv_ref, s_ref, t_ref, wf_ref, bf_ref, o_ref):
    dinv = dv_ref[...]
    agg = yp_ref[0] + yp_ref[1] + ht_ref[...]
    z = jnp.maximum(agg * dinv * s_ref[...] + t_ref[...], 0.0)
    o = z @ wf_ref[...] + bf_ref[...]
    m = jnp.max(o, axis=1, keepdims=True)
    lse = jnp.log(jnp.sum(jnp.exp(o - m), axis=1, keepdims=True)) + m
    o_ref[...] = o - lse


_GRID = 10
_BLK = N // _GRID


def _row_spec(width):
    return pl.BlockSpec((_BLK, width), lambda i: (i, 0))


def _full_spec(r, cdim):
    return pl.BlockSpec((r, cdim), lambda i: (0, 0))


def kernel(x, edge_index, W_proj0, b_proj0, W_proj1, b_proj1, alpha, W1, b1, g1, be1, W2, b2, g2, be2, Wf, bf):
    aw = jax.nn.softmax(alpha)
    Wc = jnp.concatenate([aw[0] * W_proj0, aw[1] * W_proj1], axis=0)
    bc = (aw[0] * b_proj0 + aw[1] * b_proj1).reshape(1, H)
    gs1 = (g1 / jnp.sqrt(1.0 + EPS)).reshape(1, H)
    t1 = (b1 * gs1[0] + be1).reshape(1, H)
    gs2 = (g2 / jnp.sqrt(1.0 + EPS)).reshape(1, H)
    t2 = (b2 * gs2[0] + be2).reshape(1, H)
    bf2 = bf.reshape(1, OUT)
    src_r = edge_index[0].reshape(NW, NG, CPG, K)
    dst_r = edge_index[1].reshape(NW, NG, CPG, K)
    dst1d = edge_index[1]
    zbuf = jnp.zeros((ZB, H), jnp.float32)

    dp = _deg_kernel(dst1d)

    dv = pl.pallas_call(
        _dinv_body,
        out_shape=jax.ShapeDtypeStruct((N, 1), jnp.float32),
        grid=(1,),
        in_specs=[pl.BlockSpec((NW, N), lambda i: (0, 0))],
        out_specs=pl.BlockSpec((N, 1), lambda i: (0, 0)),
    )(dp)

    ht1 = pl.pallas_call(
        _proj_body,
        out_shape=jax.ShapeDtypeStruct((N, H), jnp.float32),
        grid=(_GRID,),
        in_specs=[
            _row_spec(H),
            _full_spec(H, H),
            _full_spec(1, H),
            _full_spec(H, H),
            _row_spec(1),
        ],
        out_specs=_row_spec(H),
    )(x, Wc, bc, W1, dv)

    yp1 = _scatter_kernel(ht1, src_r, dst_r, zbuf)

    ht2 = pl.pallas_call(
        _mid_body,
        out_shape=jax.ShapeDtypeStruct((N, H), jnp.float32),
        grid=(_GRID,),
        in_specs=[
            pl.BlockSpec((NSC, _BLK, H), lambda i: (0, i, 0)),
            _row_spec(H),
            _row_spec(1),
            _full_spec(1, H),
            _full_spec(1, H),
            _full_spec(H, H),
        ],
        out_specs=_row_spec(H),
    )(yp1, ht1, dv, gs1, t1, W2)

    yp2 = _scatter_kernel(ht2, src_r, dst_r, zbuf)

    out = pl.pallas_call(
        _final_body,
        out_shape=jax.ShapeDtypeStruct((N, OUT), jnp.float32),
        grid=(_GRID,),
        in_specs=[
            pl.BlockSpec((NSC, _BLK, H), lambda i: (0, i, 0)),
            _row_spec(H),
            _row_spec(1),
            _full_spec(1, H),
            _full_spec(1, H),
            _full_spec(H, OUT),
            _full_spec(1, OUT),
        ],
        out_specs=pl.BlockSpec((_BLK, OUT), lambda i: (i, 0)),
    )(yp2, ht2, dv, gs2, t2, Wf, bf2)
    return out


# speedup vs baseline: 30.2801x; 1.0120x over previous
import functools

import jax
import jax.numpy as jnp
from jax import lax
from jax.experimental import pallas as pl
from jax.experimental.pallas import tpu as pltpu
from jax.experimental.pallas import tpu_sc as plsc

N = 10000
E = 320000
H = 128
OUT = 64
EPS = 1e-5

NSC = 2
NTEC = 16
NW = NSC * NTEC
K = 100
NCH = E // (NW * K)
CPG = 20
NG = NCH // CPG
RING = 3
NP = 10240
ZR = NP // NTEC
ZB = 80
EPT = E // NW

_mesh = plsc.VectorSubcoreMesh(core_axis_name="c", subcore_axis_name="s")


@functools.partial(
    pl.kernel,
    mesh=_mesh,
    out_type=jax.ShapeDtypeStruct((NW, N), jnp.float32),
    scratch_types=[
        pltpu.VMEM((EPT,), jnp.int32),
        pltpu.VMEM((N,), jnp.float32),
    ],
    compiler_params=pltpu.CompilerParams(needs_layout_passes=False),
)
def _deg_kernel(dst_hbm, out_hbm, dv, hist):
    c = lax.axis_index("c")
    s = lax.axis_index("s")
    w = c * NTEC + s
    pltpu.sync_copy(dst_hbm.at[pl.ds(w * EPT, EPT)], dv)
    zeros = jnp.zeros((16,), jnp.float32)

    def zbody(i, _):
        hist[pl.ds(i * 16, 16)] = zeros
        return 0

    lax.fori_loop(0, N // 16, zbody, 0, unroll=8)
    ones = jnp.ones((16,), jnp.float32)

    def body(i, _):
        idx = dv[pl.ds(i * 16, 16)]
        plsc.addupdate_scatter(hist, [idx], ones)
        return 0

    lax.fori_loop(0, EPT // 16, body, 0, unroll=8)
    pltpu.sync_copy(hist, out_hbm.at[w])


@functools.partial(
    pl.kernel,
    mesh=_mesh,
    out_type=jax.ShapeDtypeStruct((NSC, NP, H), jnp.float32),
    scratch_types=[
        pltpu.VMEM((CPG, K), jnp.int32),
        pltpu.VMEM((CPG, K), jnp.int32),
        pltpu.VMEM((RING, K, H), jnp.float32),
        pltpu.VMEM_SHARED((NP, H), jnp.float32),
        pltpu.SemaphoreType.DMA((RING,)),
        pltpu.SemaphoreType.DMA((RING,)),
    ],
    compiler_params=pltpu.CompilerParams(needs_layout_passes=False),
)
def _scatter_kernel(table_hbm, src_hbm, dst_hbm, z_hbm, out_hbm, src_v, dst_v, rows, acc, gsem, ssem):
    c = lax.axis_index("c")
    s = lax.axis_index("s")
    w = c * NTEC + s

    pltpu.sync_copy(z_hbm, rows.at[0, pl.ds(0, ZB)])
    for k in range(ZR // ZB):
        pltpu.sync_copy(rows.at[0, pl.ds(0, ZB)], acc.at[pl.ds(s * ZR + k * ZB, ZB)])
    plsc.subcore_barrier()

    def _gather(j, buf):
        pltpu.async_copy(table_hbm.at[src_v.at[j]], rows.at[buf], gsem.at[buf])

    def _gwait(buf):
        pltpu.make_async_copy(table_hbm.at[src_v.at[0]], rows.at[buf], gsem.at[buf]).wait()

    def _scat(j, buf):
        pltpu.async_copy(rows.at[buf], acc.at[dst_v.at[j]], ssem.at[buf], add=True)

    def _swait(buf):
        pltpu.make_async_copy(rows.at[buf], acc.at[dst_v.at[0]], ssem.at[buf]).wait()

    def gbody(g, _):
        pltpu.sync_copy(src_hbm.at[w, g], src_v)
        pltpu.sync_copy(dst_hbm.at[w, g], dst_v)
        for b in range(RING):
            _gather(b, b)

        def jbody(j, _):
            b = lax.rem(j, RING)
            _gwait(b)
            _scat(j, b)

            @pl.when(j + RING < CPG)
            def _():
                _swait(b)
                _gather(j + RING, b)

            return 0

        lax.fori_loop(0, CPG, jbody, 0)
        for o in range(RING):
            _swait((CPG - RING + o) % RING)
        return 0

    lax.fori_loop(0, NG, gbody, 0)
    plsc.subcore_barrier()
    for k in range(ZR // ZB):
        pltpu.sync_copy(acc.at[pl.ds(s * ZR + k * ZB, ZB)], rows.at[0, pl.ds(0, ZB)])
        pltpu.sync_copy(rows.at[0, pl.ds(0, ZB)], out_hbm.at[c, pl.ds(s * ZR + k * ZB, ZB)])


def _dinv_from_blk(dp_blk):
    deg = 1.0 + jnp.sum(dp_blk[0], axis=0)
    return lax.rsqrt(jnp.maximum(deg, 1.0))[:, None]


def _proj_body(x_ref, wc_ref, bc_ref, w1_ref, dp_ref, o_ref):
    xf = x_ref[...] @ wc_ref[...] + bc_ref[...]
    h1 = xf @ w1_ref[...]
    o_ref[...] = h1 * _dinv_from_blk(dp_ref[...])


def _mid_body(yp_ref, ht_ref, dp_ref, s_ref, t_ref, w2_ref, o_ref):
    dinv = _dinv_from_blk(dp_ref[...])
    agg = yp_ref[0] + yp_ref[1] + ht_ref[...]
    z = jnp.maximum(agg * dinv * s_ref[...] + t_ref[...], 0.0)
    o_ref[...] = (z @ w2_ref[...]) * dinv


def _final_body(yp_ref, ht_ref, dp_ref, s_ref, t_ref, wf_ref, bf_ref, o_ref):
    dinv = _dinv_from_blk(dp_ref[...])
    agg = yp_ref[0] + yp_ref[1] + ht_ref[...]
    z = jnp.maximum(agg * dinv * s_ref[...] + t_ref[...], 0.0)
    o = z @ wf_ref[...] + bf_ref[...]
    m = jnp.max(o, axis=1, keepdims=True)
    lse = jnp.log(jnp.sum(jnp.exp(o - m), axis=1, keepdims=True)) + m
    o_ref[...] = o - lse


_GRID = 10
_BLK = N // _GRID


def _row_spec(width):
    return pl.BlockSpec((_BLK, width), lambda i: (i, 0))


def _full_spec(r, cdim):
    return pl.BlockSpec((r, cdim), lambda i: (0, 0))


def kernel(x, edge_index, W_proj0, b_proj0, W_proj1, b_proj1, alpha, W1, b1, g1, be1, W2, b2, g2, be2, Wf, bf):
    aw = jax.nn.softmax(alpha)
    Wc = jnp.concatenate([aw[0] * W_proj0, aw[1] * W_proj1], axis=0)
    bc = (aw[0] * b_proj0 + aw[1] * b_proj1).reshape(1, H)
    gs1 = (g1 / jnp.sqrt(1.0 + EPS)).reshape(1, H)
    t1 = (b1 * gs1[0] + be1).reshape(1, H)
    gs2 = (g2 / jnp.sqrt(1.0 + EPS)).reshape(1, H)
    t2 = (b2 * gs2[0] + be2).reshape(1, H)
    bf2 = bf.reshape(1, OUT)
    src_r = edge_index[0].reshape(NW, NG, CPG, K)
    dst_r = edge_index[1].reshape(NW, NG, CPG, K)
    dst1d = edge_index[1]
    zbuf = jnp.zeros((ZB, H), jnp.float32)

    dp = _deg_kernel(dst1d)
    dpt = dp.reshape(NW, _GRID, _BLK).transpose(1, 0, 2)
    _dp_spec = pl.BlockSpec((1, NW, _BLK), lambda i: (i, 0, 0))

    ht1 = pl.pallas_call(
        _proj_body,
        out_shape=jax.ShapeDtypeStruct((N, H), jnp.float32),
        grid=(_GRID,),
        in_specs=[
            _row_spec(H),
            _full_spec(H, H),
            _full_spec(1, H),
            _full_spec(H, H),
            _dp_spec,
        ],
        out_specs=_row_spec(H),
    )(x, Wc, bc, W1, dpt)

    yp1 = _scatter_kernel(ht1, src_r, dst_r, zbuf)

    ht2 = pl.pallas_call(
        _mid_body,
        out_shape=jax.ShapeDtypeStruct((N, H), jnp.float32),
        grid=(_GRID,),
        in_specs=[
            pl.BlockSpec((NSC, _BLK, H), lambda i: (0, i, 0)),
            _row_spec(H),
            _dp_spec,
            _full_spec(1, H),
            _full_spec(1, H),
            _full_spec(H, H),
        ],
        out_specs=_row_spec(H),
    )(yp1, ht1, dpt, gs1, t1, W2)

    yp2 = _scatter_kernel(ht2, src_r, dst_r, zbuf)

    out = pl.pallas_call(
        _final_body,
        out_shape=jax.ShapeDtypeStruct((N, OUT), jnp.float32),
        grid=(_GRID,),
        in_specs=[
            pl.BlockSpec((NSC, _BLK, H), lambda i: (0, i, 0)),
            _row_spec(H),
            _dp_spec,
            _full_spec(1, H),
            _full_spec(1, H),
            _full_spec(H, OUT),
            _full_spec(1, OUT),
        ],
        out_specs=pl.BlockSpec((_BLK, OUT), lambda i: (i, 0)),
    )(yp2, ht2, dpt, gs2, t2, Wf, bf2)
    return out
